# baseline (speedup 1.0000x reference)
import functools

import jax
import jax.numpy as jnp
from jax import lax
from jax.experimental import pallas as pl
from jax.experimental.pallas import tpu as pltpu
from jax.experimental.pallas import tpu_sc as plsc

T, D, F, E = 2048, 1024, 512, 8
BM = 256
NB = T // BM
NT = NB + E - 1
NW = 32
CHUNK = T // NW


def _silu(x):
    return x * jax.nn.sigmoid(x)


def _router_body(x_ref, wr_ref, pos_ref, meta_ref):
    x = x_ref[...]
    logits = jnp.dot(x, wr_ref[...], preferred_element_type=jnp.float32)
    idx = jnp.argmax(logits, axis=1)
    onehot = (jax.lax.broadcasted_iota(jnp.int32, (E, T), 0)
              == idx[None, :]).astype(jnp.int32)
    csum = onehot
    k = 1
    while k < T:
        csum = csum + jnp.concatenate(
            [jnp.zeros((E, k), jnp.int32), csum[:, :T - k]], axis=1)
        k *= 2
    counts = csum[:, T - 1]
    ir = jax.lax.broadcasted_iota(jnp.int32, (E, E), 0)
    ic = jax.lax.broadcasted_iota(jnp.int32, (E, E), 1)
    off = jnp.sum(jnp.where(ir < ic, counts[:, None], 0), axis=0)
    seg_hi = off + counts
    rank = jnp.sum(jnp.where(onehot == 1, csum - 1, 0), axis=0)
    base = jnp.sum(jnp.where(onehot == 1, off[:, None], 0), axis=0)
    pos_ref[...] = rank + base

    bcol = jax.lax.broadcasted_iota(jnp.int32, (NB, E), 0) * BM
    act = ((seg_hi[None, :] > bcol) & (off[None, :] < bcol + BM)
           & (counts[None, :] > 0)).astype(jnp.int32)
    srow = act
    k = 1
    while k < E:
        srow = srow + jnp.concatenate(
            [jnp.zeros((NB, k), jnp.int32), srow[:, :E - k]], axis=1)
        k *= 2
    rowtot = srow[:, E - 1:E]
    rcs = rowtot
    k = 1
    while k < NB:
        rcs = rcs + jnp.concatenate(
            [jnp.zeros((k, 1), jnp.int32), rcs[:NB - k, :]], axis=0)
        k *= 2
    s_flat = srow + (rcs - rowtot)
    nact = rcs[NB - 1, 0]

    jj = jax.lax.broadcasted_iota(jnp.int32, (NT, NB, E), 0)
    m = ((act[None] == 1) & (s_flat[None] == jj + 1)).astype(jnp.int32)
    b3 = jax.lax.broadcasted_iota(jnp.int32, (NT, NB, E), 1)
    e3 = jax.lax.broadcasted_iota(jnp.int32, (NT, NB, E), 2)
    rb = jnp.sum(m * b3, axis=(1, 2))
    ex = jnp.sum(m * e3, axis=(1, 2))
    lo = jnp.sum(m * jnp.broadcast_to(off[None, None, :], (NT, NB, E)),
                 axis=(1, 2))
    hi = jnp.sum(m * jnp.broadcast_to(seg_hi[None, None, :], (NT, NB, E)),
                 axis=(1, 2))
    pad = jax.lax.broadcasted_iota(jnp.int32, (NT,), 0) >= nact
    rb = jnp.where(pad, NB - 1, rb)
    ex = jnp.where(pad, E - 1, ex)
    lo = jnp.where(pad, 0, lo)
    hi = jnp.where(pad, 0, hi)
    meta_ref[...] = jnp.concatenate(
        [rb[None, :], ex[None, :], lo[None, :], hi[None, :]], axis=0)


def _router(x, Wr):
    return pl.pallas_call(
        _router_body,
        out_shape=(
            jax.ShapeDtypeStruct((T,), jnp.int32),
            jax.ShapeDtypeStruct((4, NT), jnp.int32),
        ),
    )(x, Wr)


@functools.cache
def _sc_kernels():
    mesh = plsc.VectorSubcoreMesh(core_axis_name="c", subcore_axis_name="s")
    scratch = [
        pltpu.VMEM((CHUNK,), jnp.int32),
        pltpu.VMEM((CHUNK, D), jnp.float32),
        pltpu.SemaphoreType.DMA,
    ]

    @functools.partial(
        pl.kernel,
        out_type=jax.ShapeDtypeStruct((T, D), jnp.float32),
        mesh=mesh,
        scratch_types=scratch,
    )
    def sc_scatter(x_hbm, pos_hbm, xs_hbm, idx_v, rows_v, sem):
        wid = lax.axis_index("s") * 2 + lax.axis_index("c")
        base = wid * CHUNK
        pltpu.sync_copy(pos_hbm.at[pl.ds(base, CHUNK)], idx_v)
        pltpu.sync_copy(x_hbm.at[pl.ds(base, CHUNK)], rows_v)
        pltpu.async_copy(rows_v, xs_hbm.at[idx_v], sem).wait()

    @functools.partial(
        pl.kernel,
        out_type=jax.ShapeDtypeStruct((T, D), jnp.float32),
        mesh=mesh,
        scratch_types=scratch,
    )
    def sc_gather(ys_hbm, pos_hbm, out_hbm, idx_v, rows_v, sem):
        wid = lax.axis_index("s") * 2 + lax.axis_index("c")
        base = wid * CHUNK
        pltpu.sync_copy(pos_hbm.at[pl.ds(base, CHUNK)], idx_v)
        pltpu.async_copy(ys_hbm.at[idx_v], rows_v, sem).wait()
        pltpu.sync_copy(rows_v, out_hbm.at[pl.ds(base, CHUNK)])

    return sc_scatter, sc_gather


def _sc_scatter(x, pos):
    return _sc_kernels()[0](x, pos)


def _sc_gather(ys, pos):
    return _sc_kernels()[1](ys, pos)


def _group_body(m_ref, xs_ref, wr_ref, wg0_ref, wg1_ref, wu0_ref, wu1_ref,
                wd0_ref, wd1_ref, sg_ref, su_ref, sd_ref, ys_ref):
    i = pl.program_id(0)
    rb = m_ref[0, i]
    lo = m_ref[2, i]
    hi = m_ref[3, i]
    x = xs_ref[...]
    logits = jnp.dot(x, wr_ref[...], preferred_element_type=jnp.float32)
    ws = jax.nn.sigmoid(jnp.max(logits, axis=1))
    xb = x.astype(jnp.bfloat16)
    x0, x1 = xb[:, :D // 2], xb[:, D // 2:]
    g = (jnp.dot(x0, wg0_ref[0].astype(jnp.bfloat16),
                 preferred_element_type=jnp.float32)
         + jnp.dot(x1, wg1_ref[0].astype(jnp.bfloat16),
                   preferred_element_type=jnp.float32))
    u = (jnp.dot(x0, wu0_ref[0].astype(jnp.bfloat16),
                 preferred_element_type=jnp.float32)
         + jnp.dot(x1, wu1_ref[0].astype(jnp.bfloat16),
                   preferred_element_type=jnp.float32))
    a = (_silu(g) * u).astype(jnp.bfloat16)
    y = (jnp.dot(a[:, :F // 2], wd0_ref[0].astype(jnp.bfloat16),
                 preferred_element_type=jnp.float32)
         + jnp.dot(a[:, F // 2:], wd1_ref[0].astype(jnp.bfloat16),
                   preferred_element_type=jnp.float32))
    sg = jnp.dot(xb, sg_ref[...].astype(jnp.bfloat16),
                 preferred_element_type=jnp.float32)
    su = jnp.dot(xb, su_ref[...].astype(jnp.bfloat16),
                 preferred_element_type=jnp.float32)
    sh = jnp.dot((_silu(sg) * su).astype(jnp.bfloat16),
                 sd_ref[...].astype(jnp.bfloat16),
                 preferred_element_type=jnp.float32)
    row = rb * BM + jax.lax.broadcasted_iota(jnp.int32, (BM, 1), 0)
    contrib = jnp.where((row >= lo) & (row < hi), ws[:, None] * y + sh, 0.0)
    prev_rb = m_ref[0, jnp.maximum(i - 1, 0)]
    first = (i == 0) | (rb != prev_rb)

    @pl.when(first)
    def _init():
        ys_ref[...] = contrib

    @pl.when(jnp.logical_not(first))
    def _acc():
        ys_ref[...] += contrib


def _grouped(meta, xs, Wr, Wg, Wu, Wd, Sg, Su, Sd):
    grid_spec = pltpu.PrefetchScalarGridSpec(
        num_scalar_prefetch=1,
        grid=(NT,),
        in_specs=[
            pl.BlockSpec((BM, D), lambda i, m: (m[0, i], 0)),
            pl.BlockSpec((D, E), lambda i, m: (0, 0)),
            pl.BlockSpec((1, D // 2, F), lambda i, m: (m[1, i], 0, 0)),
            pl.BlockSpec((1, D // 2, F), lambda i, m: (m[1, i], 1, 0)),
            pl.BlockSpec((1, D // 2, F), lambda i, m: (m[1, i], 0, 0)),
            pl.BlockSpec((1, D // 2, F), lambda i, m: (m[1, i], 1, 0)),
            pl.BlockSpec((1, F // 2, D), lambda i, m: (m[1, i], 0, 0)),
            pl.BlockSpec((1, F // 2, D), lambda i, m: (m[1, i], 1, 0)),
            pl.BlockSpec((D, F), lambda i, m: (0, 0)),
            pl.BlockSpec((D, F), lambda i, m: (0, 0)),
            pl.BlockSpec((F, D), lambda i, m: (0, 0)),
        ],
        out_specs=pl.BlockSpec((BM, D), lambda i, m: (m[0, i], 0)),
    )
    return pl.pallas_call(
        _group_body,
        grid_spec=grid_spec,
        out_shape=jax.ShapeDtypeStruct((T, D), jnp.float32),
    )(meta, xs, Wr, Wg, Wg, Wu, Wu, Wd, Wd, Sg, Su, Sd)


@jax.jit
def kernel(hidden_states, Wr, Wg, Wu, Wd, Sg, Su, Sd):
    pos, meta = _router(hidden_states, Wr)
    xs = _sc_scatter(hidden_states, pos)
    ys = _grouped(meta, xs, Wr, Wg, Wu, Wd, Sg, Su, Sd)
    return _sc_gather(ys, pos)


# speedup vs baseline: 1.0238x; 1.0238x over previous
import functools

import jax
import jax.numpy as jnp
from jax import lax
from jax.experimental import pallas as pl
from jax.experimental.pallas import tpu as pltpu
from jax.experimental.pallas import tpu_sc as plsc

T, D, F, E = 2048, 1024, 512, 8
BM = 256
NB = T // BM
NT = NB + E - 1
NW = 32
CHUNK = T // NW


def _silu(x):
    return x * jax.nn.sigmoid(x)


KD = 4


def _router_body(x_ref, wr_ref, pos_ref, meta_ref, acc_ref):
    k = pl.program_id(0)
    x = x_ref[...]
    part = jnp.dot(x, wr_ref[...], preferred_element_type=jnp.float32)

    @pl.when(k == 0)
    def _first():
        acc_ref[...] = part

    @pl.when(k > 0)
    def _acc():
        acc_ref[...] += part

    @pl.when(k == KD - 1)
    def _finish():
        _router_tail(acc_ref[...], pos_ref, meta_ref)


def _router_tail(logits, pos_ref, meta_ref):
    idx = jnp.argmax(logits, axis=1)
    onehot = (jax.lax.broadcasted_iota(jnp.int32, (E, T), 0)
              == idx[None, :]).astype(jnp.int32)
    csum = onehot
    k = 1
    while k < T:
        csum = csum + jnp.concatenate(
            [jnp.zeros((E, k), jnp.int32), csum[:, :T - k]], axis=1)
        k *= 2
    counts = csum[:, T - 1]
    ir = jax.lax.broadcasted_iota(jnp.int32, (E, E), 0)
    ic = jax.lax.broadcasted_iota(jnp.int32, (E, E), 1)
    off = jnp.sum(jnp.where(ir < ic, counts[:, None], 0), axis=0)
    seg_hi = off + counts
    rank = jnp.sum(jnp.where(onehot == 1, csum - 1, 0), axis=0)
    base = jnp.sum(jnp.where(onehot == 1, off[:, None], 0), axis=0)
    pos_ref[...] = rank + base

    bcol = jax.lax.broadcasted_iota(jnp.int32, (NB, E), 0) * BM
    act = ((seg_hi[None, :] > bcol) & (off[None, :] < bcol + BM)
           & (counts[None, :] > 0)).astype(jnp.int32)
    srow = act
    k = 1
    while k < E:
        srow = srow + jnp.concatenate(
            [jnp.zeros((NB, k), jnp.int32), srow[:, :E - k]], axis=1)
        k *= 2
    rowtot = srow[:, E - 1:E]
    rcs = rowtot
    k = 1
    while k < NB:
        rcs = rcs + jnp.concatenate(
            [jnp.zeros((k, 1), jnp.int32), rcs[:NB - k, :]], axis=0)
        k *= 2
    s_flat = srow + (rcs - rowtot)
    nact = rcs[NB - 1, 0]

    jj = jax.lax.broadcasted_iota(jnp.int32, (NT, NB, E), 0)
    m = ((act[None] == 1) & (s_flat[None] == jj + 1)).astype(jnp.int32)
    b3 = jax.lax.broadcasted_iota(jnp.int32, (NT, NB, E), 1)
    e3 = jax.lax.broadcasted_iota(jnp.int32, (NT, NB, E), 2)
    rb = jnp.sum(m * b3, axis=(1, 2))
    ex = jnp.sum(m * e3, axis=(1, 2))
    lo = jnp.sum(m * jnp.broadcast_to(off[None, None, :], (NT, NB, E)),
                 axis=(1, 2))
    hi = jnp.sum(m * jnp.broadcast_to(seg_hi[None, None, :], (NT, NB, E)),
                 axis=(1, 2))
    pad = jax.lax.broadcasted_iota(jnp.int32, (NT,), 0) >= nact
    rb = jnp.where(pad, NB - 1, rb)
    ex = jnp.where(pad, E - 1, ex)
    lo = jnp.where(pad, 0, lo)
    hi = jnp.where(pad, 0, hi)
    meta_ref[...] = jnp.concatenate(
        [rb[None, :], ex[None, :], lo[None, :], hi[None, :]], axis=0)


def _router(x, Wr):
    return pl.pallas_call(
        _router_body,
        grid=(KD,),
        in_specs=[
            pl.BlockSpec((T, D // KD), lambda k: (0, k)),
            pl.BlockSpec((D // KD, E), lambda k: (k, 0)),
        ],
        out_specs=(
            pl.BlockSpec((T,), lambda k: (0,)),
            pl.BlockSpec((4, NT), lambda k: (0, 0)),
        ),
        out_shape=(
            jax.ShapeDtypeStruct((T,), jnp.int32),
            jax.ShapeDtypeStruct((4, NT), jnp.int32),
        ),
        scratch_shapes=[pltpu.VMEM((T, E), jnp.float32)],
    )(x, Wr)


@functools.cache
def _sc_kernels():
    mesh = plsc.VectorSubcoreMesh(core_axis_name="c", subcore_axis_name="s")

    @functools.partial(
        pl.kernel,
        out_type=jax.ShapeDtypeStruct((T, D), jnp.float32),
        mesh=mesh,
        scratch_types=[
            pltpu.VMEM((CHUNK,), jnp.int32),
            pltpu.VMEM((CHUNK, D), jnp.float32),
            pltpu.SemaphoreType.DMA,
        ],
    )
    def sc_scatter(x_hbm, pos_hbm, xs_hbm, idx_v, rows_v, sem):
        wid = lax.axis_index("s") * 2 + lax.axis_index("c")
        base = wid * CHUNK
        pltpu.sync_copy(pos_hbm.at[pl.ds(base, CHUNK)], idx_v)
        pltpu.sync_copy(x_hbm.at[pl.ds(base, CHUNK)], rows_v)
        pltpu.async_copy(rows_v, xs_hbm.at[idx_v], sem).wait()

    @functools.partial(
        pl.kernel,
        out_type=jax.ShapeDtypeStruct((T, D), jnp.float32),
        mesh=mesh,
        scratch_types=[
            pltpu.VMEM((CHUNK,), jnp.int32),
            pltpu.VMEM((CHUNK, D), jnp.float32),
            pltpu.SemaphoreType.DMA,
        ],
    )
    def sc_gather(ys_hbm, pos_hbm, out_hbm, idx_v, rows_v, sem):
        wid = lax.axis_index("s") * 2 + lax.axis_index("c")
        base = wid * CHUNK
        pltpu.sync_copy(pos_hbm.at[pl.ds(base, CHUNK)], idx_v)
        pltpu.async_copy(ys_hbm.at[idx_v], rows_v, sem).wait()
        pltpu.sync_copy(rows_v, out_hbm.at[pl.ds(base, CHUNK)])

    return sc_scatter, sc_gather


def _sc_scatter(x, pos):
    return _sc_kernels()[0](x, pos)


def _sc_gather(ys, pos):
    return _sc_kernels()[1](ys, pos)


def _group_body(m_ref, xs_ref, wr_ref, wg0_ref, wg1_ref, wu0_ref, wu1_ref,
                wd0_ref, wd1_ref, sg_ref, su_ref, sd_ref, ys_ref):
    i = pl.program_id(0)
    rb = m_ref[0, i]
    lo = m_ref[2, i]
    hi = m_ref[3, i]
    x = xs_ref[...]
    logits = jnp.dot(x, wr_ref[...], preferred_element_type=jnp.float32)
    xb = x.astype(jnp.bfloat16)
    ws = jax.nn.sigmoid(jnp.max(logits, axis=1))
    x0, x1 = xb[:, :D // 2], xb[:, D // 2:]
    g = (jnp.dot(x0, wg0_ref[0].astype(jnp.bfloat16),
                 preferred_element_type=jnp.float32)
         + jnp.dot(x1, wg1_ref[0].astype(jnp.bfloat16),
                   preferred_element_type=jnp.float32))
    u = (jnp.dot(x0, wu0_ref[0].astype(jnp.bfloat16),
                 preferred_element_type=jnp.float32)
         + jnp.dot(x1, wu1_ref[0].astype(jnp.bfloat16),
                   preferred_element_type=jnp.float32))
    a = (_silu(g) * u).astype(jnp.bfloat16)
    y = (jnp.dot(a[:, :F // 2], wd0_ref[0].astype(jnp.bfloat16),
                 preferred_element_type=jnp.float32)
         + jnp.dot(a[:, F // 2:], wd1_ref[0].astype(jnp.bfloat16),
                   preferred_element_type=jnp.float32))
    row = rb * BM + jax.lax.broadcasted_iota(jnp.int32, (BM, 1), 0)
    routed = jnp.where((row >= lo) & (row < hi), ws[:, None] * y, 0.0)
    prev_rb = m_ref[0, jnp.maximum(i - 1, 0)]
    first = (i == 0) | (rb != prev_rb)

    @pl.when(first)
    def _init():
        sg = jnp.dot(xb, sg_ref[...].astype(jnp.bfloat16),
                     preferred_element_type=jnp.float32)
        su = jnp.dot(xb, su_ref[...].astype(jnp.bfloat16),
                     preferred_element_type=jnp.float32)
        sh = jnp.dot((_silu(sg) * su).astype(jnp.bfloat16),
                     sd_ref[...].astype(jnp.bfloat16),
                     preferred_element_type=jnp.float32)
        ys_ref[...] = routed + sh

    @pl.when(jnp.logical_not(first))
    def _acc():
        ys_ref[...] += routed


def _grouped(meta, xs, Wr, Wg, Wu, Wd, Sg, Su, Sd):
    grid_spec = pltpu.PrefetchScalarGridSpec(
        num_scalar_prefetch=1,
        grid=(NT,),
        in_specs=[
            pl.BlockSpec((BM, D), lambda i, m: (m[0, i], 0)),
            pl.BlockSpec((D, E), lambda i, m: (0, 0)),
            pl.BlockSpec((1, D // 2, F), lambda i, m: (m[1, i], 0, 0)),
            pl.BlockSpec((1, D // 2, F), lambda i, m: (m[1, i], 1, 0)),
            pl.BlockSpec((1, D // 2, F), lambda i, m: (m[1, i], 0, 0)),
            pl.BlockSpec((1, D // 2, F), lambda i, m: (m[1, i], 1, 0)),
            pl.BlockSpec((1, F // 2, D), lambda i, m: (m[1, i], 0, 0)),
            pl.BlockSpec((1, F // 2, D), lambda i, m: (m[1, i], 1, 0)),
            pl.BlockSpec((D, F), lambda i, m: (0, 0)),
            pl.BlockSpec((D, F), lambda i, m: (0, 0)),
            pl.BlockSpec((F, D), lambda i, m: (0, 0)),
        ],
        out_specs=pl.BlockSpec((BM, D), lambda i, m: (m[0, i], 0)),
    )
    return pl.pallas_call(
        _group_body,
        grid_spec=grid_spec,
        out_shape=jax.ShapeDtypeStruct((T, D), jnp.float32),
    )(meta, xs, Wr, Wg, Wg, Wu, Wu, Wd, Wd, Sg, Su, Sd)


@jax.jit
def kernel(hidden_states, Wr, Wg, Wu, Wd, Sg, Su, Sd):
    pos, meta = _router(hidden_states, Wr)
    xs = _sc_scatter(hidden_states, pos)
    ys = _grouped(meta, xs, Wr, Wg, Wu, Wd, Sg, Su, Sd)
    return _sc_gather(ys, pos)


# speedup vs baseline: 1.0416x; 1.0174x over previous
import functools

import jax
import jax.numpy as jnp
from jax import lax
from jax.experimental import pallas as pl
from jax.experimental.pallas import tpu as pltpu
from jax.experimental.pallas import tpu_sc as plsc

T, D, F, E = 2048, 1024, 512, 8
BM = 256
BM_LOG2 = 8
NB = T // BM
NW = 32
CHUNK = T // NW


def _silu(x):
    return x * jax.nn.sigmoid(x)


KD = 4


def _router_body(x_ref, wr_ref, pos_ref, meta_ref, acc_ref):
    k = pl.program_id(0)
    x = x_ref[...]
    part = jnp.dot(x, wr_ref[...], preferred_element_type=jnp.float32)

    @pl.when(k == 0)
    def _first():
        acc_ref[...] = part

    @pl.when(k > 0)
    def _acc():
        acc_ref[...] += part

    @pl.when(k == KD - 1)
    def _finish():
        _router_tail(acc_ref[...], pos_ref, meta_ref)


def _router_tail(logits, pos_ref, meta_ref):
    idx = jnp.argmax(logits, axis=1)
    onehot = (jax.lax.broadcasted_iota(jnp.int32, (E, T), 0)
              == idx[None, :]).astype(jnp.int32)
    csum = onehot
    k = 1
    while k < T:
        csum = csum + jnp.concatenate(
            [jnp.zeros((E, k), jnp.int32), csum[:, :T - k]], axis=1)
        k *= 2
    counts = csum[:, T - 1]
    ir = jax.lax.broadcasted_iota(jnp.int32, (E, E), 0)
    ic = jax.lax.broadcasted_iota(jnp.int32, (E, E), 1)
    off = jnp.sum(jnp.where(ir < ic, counts[:, None], 0), axis=0)
    seg_hi = off + counts
    rank = jnp.sum(jnp.where(onehot == 1, csum - 1, 0), axis=0)
    base = jnp.sum(jnp.where(onehot == 1, off[:, None], 0), axis=0)
    pos_ref[...] = rank + base

    nz = counts > 0
    blo = jnp.where(nz, jnp.right_shift(off, BM_LOG2), 0)
    bhi = jnp.where(nz, jnp.right_shift(seg_hi + (BM - 1), BM_LOG2), 0)
    meta_ref[...] = jnp.concatenate(
        [blo[None, :], bhi[None, :], off[None, :], seg_hi[None, :]], axis=0)


def _router(x, Wr):
    return pl.pallas_call(
        _router_body,
        grid=(KD,),
        in_specs=[
            pl.BlockSpec((T, D // KD), lambda k: (0, k)),
            pl.BlockSpec((D // KD, E), lambda k: (k, 0)),
        ],
        out_specs=(
            pl.BlockSpec((T,), lambda k: (0,)),
            pl.BlockSpec((4, E), lambda k: (0, 0)),
        ),
        out_shape=(
            jax.ShapeDtypeStruct((T,), jnp.int32),
            jax.ShapeDtypeStruct((4, E), jnp.int32),
        ),
        scratch_shapes=[pltpu.VMEM((T, E), jnp.float32)],
    )(x, Wr)


@functools.cache
def _sc_kernels():
    mesh = plsc.VectorSubcoreMesh(core_axis_name="c", subcore_axis_name="s")

    @functools.partial(
        pl.kernel,
        out_type=jax.ShapeDtypeStruct((T, D), jnp.float32),
        mesh=mesh,
        scratch_types=[
            pltpu.VMEM((CHUNK,), jnp.int32),
            pltpu.VMEM((CHUNK, D), jnp.float32),
            pltpu.SemaphoreType.DMA,
        ],
    )
    def sc_scatter(x_hbm, pos_hbm, xs_hbm, idx_v, rows_v, sem):
        wid = lax.axis_index("s") * 2 + lax.axis_index("c")
        base = wid * CHUNK
        pltpu.sync_copy(pos_hbm.at[pl.ds(base, CHUNK)], idx_v)
        pltpu.sync_copy(x_hbm.at[pl.ds(base, CHUNK)], rows_v)
        pltpu.async_copy(rows_v, xs_hbm.at[idx_v], sem).wait()

    @functools.partial(
        pl.kernel,
        out_type=jax.ShapeDtypeStruct((T, D), jnp.float32),
        mesh=mesh,
        scratch_types=[
            pltpu.VMEM((CHUNK,), jnp.int32),
            pltpu.VMEM((CHUNK, D), jnp.float32),
            pltpu.SemaphoreType.DMA,
        ],
    )
    def sc_gather(ys_hbm, pos_hbm, out_hbm, idx_v, rows_v, sem):
        wid = lax.axis_index("s") * 2 + lax.axis_index("c")
        base = wid * CHUNK
        pltpu.sync_copy(pos_hbm.at[pl.ds(base, CHUNK)], idx_v)
        pltpu.async_copy(ys_hbm.at[idx_v], rows_v, sem).wait()
        pltpu.sync_copy(rows_v, out_hbm.at[pl.ds(base, CHUNK)])

    return sc_scatter, sc_gather


def _sc_scatter(x, pos):
    return _sc_kernels()[0](x, pos)


def _sc_gather(ys, pos):
    return _sc_kernels()[1](ys, pos)


def _group_body(m_ref, xs_ref, wr_ref, wg_ref, wu_ref, wd_ref,
                sg_ref, su_ref, sd_ref, ys_ref):
    e = pl.program_id(0)

    @pl.when(e == 0)
    def _shared_init():
        sgw = sg_ref[...].astype(jnp.bfloat16)
        suw = su_ref[...].astype(jnp.bfloat16)
        sdw = sd_ref[...].astype(jnp.bfloat16)
        for b in range(NB):
            xb = xs_ref[b * BM:(b + 1) * BM, :].astype(jnp.bfloat16)
            sg = jnp.dot(xb, sgw, preferred_element_type=jnp.float32)
            su = jnp.dot(xb, suw, preferred_element_type=jnp.float32)
            ys_ref[b * BM:(b + 1) * BM, :] = jnp.dot(
                (_silu(sg) * su).astype(jnp.bfloat16), sdw,
                preferred_element_type=jnp.float32)

    lo = m_ref[2, e]
    hi = m_ref[3, e]
    wr = wr_ref[...]
    wg = wg_ref[0].astype(jnp.bfloat16)
    wu = wu_ref[0].astype(jnp.bfloat16)
    wd = wd_ref[0].astype(jnp.bfloat16)

    def tile(bj, carry):
        start = pl.multiple_of(bj * BM, BM)
        x = xs_ref[pl.ds(start, BM), :]
        logits = jnp.dot(x, wr, preferred_element_type=jnp.float32)
        ws = jax.nn.sigmoid(jnp.max(logits, axis=1))
        xb = x.astype(jnp.bfloat16)
        g = jnp.dot(xb, wg, preferred_element_type=jnp.float32)
        u = jnp.dot(xb, wu, preferred_element_type=jnp.float32)
        y = jnp.dot((_silu(g) * u).astype(jnp.bfloat16), wd,
                    preferred_element_type=jnp.float32)
        row = start + jax.lax.broadcasted_iota(jnp.int32, (BM, 1), 0)
        routed = jnp.where((row >= lo) & (row < hi), ws[:, None] * y, 0.0)
        ys_ref[pl.ds(start, BM), :] += routed
        return carry

    lax.fori_loop(m_ref[0, e], m_ref[1, e], tile, 0)


def _grouped(meta, xs, Wr, Wg, Wu, Wd, Sg, Su, Sd):
    grid_spec = pltpu.PrefetchScalarGridSpec(
        num_scalar_prefetch=1,
        grid=(E,),
        in_specs=[
            pl.BlockSpec((T, D), lambda e, m: (0, 0)),
            pl.BlockSpec((D, E), lambda e, m: (0, 0)),
            pl.BlockSpec((1, D, F), lambda e, m: (e, 0, 0)),
            pl.BlockSpec((1, D, F), lambda e, m: (e, 0, 0)),
            pl.BlockSpec((1, F, D), lambda e, m: (e, 0, 0)),
            pl.BlockSpec((D, F), lambda e, m: (0, 0)),
            pl.BlockSpec((D, F), lambda e, m: (0, 0)),
            pl.BlockSpec((F, D), lambda e, m: (0, 0)),
        ],
        out_specs=pl.BlockSpec((T, D), lambda e, m: (0, 0)),
    )
    return pl.pallas_call(
        _group_body,
        grid_spec=grid_spec,
        out_shape=jax.ShapeDtypeStruct((T, D), jnp.float32),
    )(meta, xs, Wr, Wg, Wu, Wd, Sg, Su, Sd)


@jax.jit
def kernel(hidden_states, Wr, Wg, Wu, Wd, Sg, Su, Sd):
    pos, meta = _router(hidden_states, Wr)
    xs = _sc_scatter(hidden_states, pos)
    ys = _grouped(meta, xs, Wr, Wg, Wu, Wd, Sg, Su, Sd)
    return _sc_gather(ys, pos)


# speedup vs baseline: 1.0554x; 1.0132x over previous
import functools

import jax
import jax.numpy as jnp
from jax import lax
from jax.experimental import pallas as pl
from jax.experimental.pallas import tpu as pltpu
from jax.experimental.pallas import tpu_sc as plsc

T, D, F, E = 2048, 1024, 512, 8
BM = 256
BM_LOG2 = 8
NB = T // BM
NW = 32
CHUNK = T // NW


def _silu(x):
    return x * jax.nn.sigmoid(x)


KD = 4


def _router_body(x_ref, wr_ref, pos_ref, meta_ref, acc_ref):
    k = pl.program_id(0)
    x = x_ref[...]
    part = jnp.dot(x, wr_ref[...], preferred_element_type=jnp.float32)

    @pl.when(k == 0)
    def _first():
        acc_ref[...] = part

    @pl.when(k > 0)
    def _acc():
        acc_ref[...] += part

    @pl.when(k == KD - 1)
    def _finish():
        _router_tail(acc_ref[...], pos_ref, meta_ref)


def _router_tail(logits, pos_ref, meta_ref):
    idx = jnp.argmax(logits, axis=1)
    onehot = (jax.lax.broadcasted_iota(jnp.int32, (E, T), 0)
              == idx[None, :]).astype(jnp.int32)
    csum = onehot
    k = 1
    while k < T:
        csum = csum + jnp.concatenate(
            [jnp.zeros((E, k), jnp.int32), csum[:, :T - k]], axis=1)
        k *= 2
    counts = csum[:, T - 1]
    ir = jax.lax.broadcasted_iota(jnp.int32, (E, E), 0)
    ic = jax.lax.broadcasted_iota(jnp.int32, (E, E), 1)
    off = jnp.sum(jnp.where(ir < ic, counts[:, None], 0), axis=0)
    seg_hi = off + counts
    rank = jnp.sum(jnp.where(onehot == 1, csum - 1, 0), axis=0)
    base = jnp.sum(jnp.where(onehot == 1, off[:, None], 0), axis=0)
    pos_ref[...] = rank + base

    nz = counts > 0
    blo = jnp.where(nz, jnp.right_shift(off, BM_LOG2), 0)
    bhi = jnp.where(nz, jnp.right_shift(seg_hi + (BM - 1), BM_LOG2), 0)
    meta_ref[...] = jnp.concatenate(
        [blo[None, :], bhi[None, :], off[None, :], seg_hi[None, :]], axis=0)


def _router(x, Wr):
    return pl.pallas_call(
        _router_body,
        grid=(KD,),
        in_specs=[
            pl.BlockSpec((T, D // KD), lambda k: (0, k)),
            pl.BlockSpec((D // KD, E), lambda k: (k, 0)),
        ],
        out_specs=(
            pl.BlockSpec((T,), lambda k: (0,)),
            pl.BlockSpec((4, E), lambda k: (0, 0)),
        ),
        out_shape=(
            jax.ShapeDtypeStruct((T,), jnp.int32),
            jax.ShapeDtypeStruct((4, E), jnp.int32),
        ),
        scratch_shapes=[pltpu.VMEM((T, E), jnp.float32)],
    )(x, Wr)


@functools.cache
def _sc_kernels():
    mesh = plsc.VectorSubcoreMesh(core_axis_name="c", subcore_axis_name="s")

    @functools.partial(
        pl.kernel,
        out_type=jax.ShapeDtypeStruct((T, D), jnp.float32),
        mesh=mesh,
        scratch_types=[
            pltpu.VMEM((CHUNK,), jnp.int32),
            pltpu.VMEM((CHUNK, D), jnp.float32),
            pltpu.SemaphoreType.DMA,
        ],
    )
    def sc_scatter(x_hbm, pos_hbm, xs_hbm, idx_v, rows_v, sem):
        wid = lax.axis_index("s") * 2 + lax.axis_index("c")
        base = wid * CHUNK
        pltpu.sync_copy(pos_hbm.at[pl.ds(base, CHUNK)], idx_v)
        pltpu.sync_copy(x_hbm.at[pl.ds(base, CHUNK)], rows_v)
        pltpu.async_copy(rows_v, xs_hbm.at[idx_v], sem).wait()

    @functools.partial(
        pl.kernel,
        out_type=jax.ShapeDtypeStruct((T, D), jnp.float32),
        mesh=mesh,
        scratch_types=[
            pltpu.VMEM((CHUNK,), jnp.int32),
            pltpu.VMEM((CHUNK, D), jnp.float32),
            pltpu.SemaphoreType.DMA,
        ],
    )
    def sc_gather(ys_hbm, pos_hbm, out_hbm, idx_v, rows_v, sem):
        wid = lax.axis_index("s") * 2 + lax.axis_index("c")
        base = wid * CHUNK
        pltpu.sync_copy(pos_hbm.at[pl.ds(base, CHUNK)], idx_v)
        pltpu.async_copy(ys_hbm.at[idx_v], rows_v, sem).wait()
        pltpu.sync_copy(rows_v, out_hbm.at[pl.ds(base, CHUNK)])

    return sc_scatter, sc_gather


def _sc_scatter(x, pos):
    return _sc_kernels()[0](x, pos)


def _sc_gather(ys, pos):
    return _sc_kernels()[1](ys, pos)


def _group_body(m_ref, xs_ref, wr_ref, wg_ref, wu_ref, wd_ref,
                sg_ref, su_ref, sd_ref, ys_ref):
    i = pl.program_id(0)

    @pl.when(i == 0)
    def _shared_init():
        sgw = sg_ref[...].astype(jnp.bfloat16)
        suw = su_ref[...].astype(jnp.bfloat16)
        sdw = sd_ref[...].astype(jnp.bfloat16)
        for b in range(NB):
            xb = xs_ref[b * BM:(b + 1) * BM, :].astype(jnp.bfloat16)
            sg = jnp.dot(xb, sgw, preferred_element_type=jnp.float32)
            su = jnp.dot(xb, suw, preferred_element_type=jnp.float32)
            ys_ref[b * BM:(b + 1) * BM, :] = jnp.dot(
                (_silu(sg) * su).astype(jnp.bfloat16), sdw,
                preferred_element_type=jnp.float32)

    @pl.when(i > 0)
    def _expert():
        e = i - 1
        lo = m_ref[2, e]
        hi = m_ref[3, e]
        wr = wr_ref[...]
        wg = wg_ref[0].astype(jnp.bfloat16)
        wu = wu_ref[0].astype(jnp.bfloat16)
        wd = wd_ref[0].astype(jnp.bfloat16)

        def tile(bj, carry):
            start = pl.multiple_of(bj * BM, BM)
            x = xs_ref[pl.ds(start, BM), :]
            logits = jnp.dot(x, wr, preferred_element_type=jnp.float32)
            ws = jax.nn.sigmoid(jnp.max(logits, axis=1))
            xb = x.astype(jnp.bfloat16)
            g = jnp.dot(xb, wg, preferred_element_type=jnp.float32)
            u = jnp.dot(xb, wu, preferred_element_type=jnp.float32)
            y = jnp.dot((_silu(g) * u).astype(jnp.bfloat16), wd,
                        preferred_element_type=jnp.float32)
            row = start + jax.lax.broadcasted_iota(jnp.int32, (BM, 1), 0)
            routed = jnp.where((row >= lo) & (row < hi), ws[:, None] * y, 0.0)
            ys_ref[pl.ds(start, BM), :] += routed
            return carry

        lax.fori_loop(m_ref[0, e], m_ref[1, e], tile, 0)


def _grouped(meta, xs, Wr, Wg, Wu, Wd, Sg, Su, Sd):
    grid_spec = pltpu.PrefetchScalarGridSpec(
        num_scalar_prefetch=1,
        grid=(E + 1,),
        in_specs=[
            pl.BlockSpec((T, D), lambda i, m: (0, 0)),
            pl.BlockSpec((D, E), lambda i, m: (0, 0)),
            pl.BlockSpec((1, D, F), lambda i, m: (jnp.maximum(i - 1, 0), 0, 0)),
            pl.BlockSpec((1, D, F), lambda i, m: (jnp.maximum(i - 1, 0), 0, 0)),
            pl.BlockSpec((1, F, D), lambda i, m: (jnp.maximum(i - 1, 0), 0, 0)),
            pl.BlockSpec((D, F), lambda i, m: (0, 0)),
            pl.BlockSpec((D, F), lambda i, m: (0, 0)),
            pl.BlockSpec((F, D), lambda i, m: (0, 0)),
        ],
        out_specs=pl.BlockSpec((T, D), lambda i, m: (0, 0)),
    )
    return pl.pallas_call(
        _group_body,
        grid_spec=grid_spec,
        out_shape=jax.ShapeDtypeStruct((T, D), jnp.float32),
    )(meta, xs, Wr, Wg, Wu, Wd, Sg, Su, Sd)


@jax.jit
def kernel(hidden_states, Wr, Wg, Wu, Wd, Sg, Su, Sd):
    pos, meta = _router(hidden_states, Wr)
    xs = _sc_scatter(hidden_states, pos)
    ys = _grouped(meta, xs, Wr, Wg, Wu, Wd, Sg, Su, Sd)
    return _sc_gather(ys, pos)
